# baseline (speedup 1.0000x reference)
import functools

import numpy as np
import jax
import jax.numpy as jnp
from jax import lax
from jax.experimental import pallas as pl
from jax.experimental.pallas import tpu as pltpu
from jax.experimental.pallas import tpu_sc as plsc

_B, _L = 8, 512
_LH = _L - 1
_N_ENTITY, _N_REL, _N_GROUPS = 2000, 50, 100
_HIDDEN = 128
_D_MODEL, _D_TIME = 128, 32
_N_HEAD, _N_LAYERS = 2, 2
_GP_DIM = 64
_MHA_HD = 32
_NTOK = _B * _L

_NW = 32
_CHUNK = _NTOK // _NW

_OCHUNKS = ((0, 128), (128, 128), (256, 128), (384, 127))
_N_OCHUNK = len(_OCHUNKS)


def _sc_gather_body(subs_hbm, marks_hbm, objs_hbm, gmap_hbm, evemb_hbm,
                    subemb_hbm, relemb_hbm, subs0_hbm, marks0_hbm,
                    gid_out, x_out, semb_out, remb_out,
                    ia_v, ib_v, rows_v, idx8_v, rows8_v, sem):
    wid = lax.axis_index("s") * 2 + lax.axis_index("c")
    base = wid * _CHUNK
    sl = pl.ds(base, _CHUNK)

    pltpu.sync_copy(subs_hbm.at[sl], ia_v)
    pltpu.sync_copy(marks_hbm.at[sl], ib_v)
    for i in range(_CHUNK // 16):
        v = pl.ds(i * 16, 16)
        ib_v[v] = ia_v[v] * _N_REL + ib_v[v]
    pltpu.async_copy(gmap_hbm.at[ib_v], ia_v, sem).wait()
    pltpu.sync_copy(ia_v, gid_out.at[sl])

    pltpu.sync_copy(objs_hbm.at[sl], ia_v)
    pltpu.async_copy(evemb_hbm.at[ia_v], rows_v, sem).wait()
    pltpu.sync_copy(rows_v, x_out.at[sl])

    @pl.when(wid == 0)
    def _():
        pltpu.sync_copy(subs0_hbm, idx8_v)
        pltpu.async_copy(subemb_hbm.at[idx8_v], rows8_v, sem).wait()
        pltpu.sync_copy(rows8_v, semb_out)

    @pl.when(wid == 1)
    def _():
        pltpu.sync_copy(marks0_hbm, idx8_v)
        pltpu.async_copy(relemb_hbm.at[idx8_v], rows8_v, sem).wait()
        pltpu.sync_copy(rows8_v, remb_out)


_sc_gather_cache = []


def _sc_gather(*args):
    if not _sc_gather_cache:
        _sc_gather_cache.append(_make_sc_gather())
    return _sc_gather_cache[0](*args)


def _make_sc_gather():
    return functools.partial(
        pl.kernel,
        out_type=(
        jax.ShapeDtypeStruct((_NTOK,), jnp.int32),
        jax.ShapeDtypeStruct((_NTOK, _D_MODEL), jnp.float32),
            jax.ShapeDtypeStruct((_B, _HIDDEN), jnp.float32),
            jax.ShapeDtypeStruct((_B, _HIDDEN), jnp.float32),
        ),
        mesh=plsc.VectorSubcoreMesh(core_axis_name="c", subcore_axis_name="s"),
        scratch_types=(
            pltpu.VMEM((_CHUNK,), jnp.int32),
            pltpu.VMEM((_CHUNK,), jnp.int32),
            pltpu.VMEM((_CHUNK, _D_MODEL), jnp.float32),
            pltpu.VMEM((_B,), jnp.int32),
            pltpu.VMEM((_B, _HIDDEN), jnp.float32),
            pltpu.SemaphoreType.DMA,
        ),
    )(_sc_gather_body)


def _mm(a, b):
    return lax.dot_general(a, b, (((1,), (0,)), ((), ())),
                           preferred_element_type=jnp.float32)


def _mm_t(a, b):
    return lax.dot_general(a, b, (((1,), (1,)), ((), ())),
                           preferred_element_type=jnp.float32)


def _mm_tl(a, b):
    return lax.dot_general(a, b, (((0,), (0,)), ((), ())),
                           preferred_element_type=jnp.float32)


def _bf(a):
    return a.astype(jnp.bfloat16)


def _mmb(a, b):
    return _mm(_bf(a), _bf(b))


def _mmb_t(a, b):
    return _mm_t(_bf(a), _bf(b))


_PAIR = 1
_NSTEP = _B // _PAIR


def _tc_body(*refs):
    (x_ref, tc_ref, gid_ref, msk_ref, se_ref, re_ref) = refs[:6]
    wrefs = refs[6:18]
    (gpw_ref, gpb_ref,
     mwq_ref, mwk_ref, mwv_ref, mwo_ref,
     mbq_ref, mbk_ref, mbv_ref, mbo_ref,
     fw1_ref, fb1_ref, fw2_ref, fb2_ref,
     l1g_ref, l1b_ref, l2g_ref, l2b_ref,
     mgw_ref, mgb_ref, intw_ref, intb_ref) = refs[18:40]
    out_ref = refs[40]
    scr = refs[41:41 + _PAIR * _N_OCHUNK]
    sems = refs[41 + _PAIR * _N_OCHUNK]
    b = pl.program_id(0)

    half = _D_TIME // 2
    ci3 = lax.broadcasted_iota(jnp.int32, (3, 3 * _D_TIME), 1)
    ri3 = lax.broadcasted_iota(jnp.int32, (3, 3 * _D_TIME), 0)
    kk = ci3 & (_D_TIME - 1)
    k16 = jnp.where(kk < half, kk, kk - half).astype(jnp.float32)
    dvv = jnp.exp(-k16 * (np.log(10000.0) / (half - 1)))
    sel = jnp.where(lax.shift_right_logical(ci3, 5) == ri3, dvv, 0.0)
    ph96 = jnp.where(kk < half, np.float32(np.pi / 2), 0.0)[0:1, :]

    ri = lax.broadcasted_iota(jnp.int32, (_LH, _LH), 0)
    ci = lax.broadcasted_iota(jnp.int32, (_LH, _LH), 1)
    causal = ci <= ri
    scale = 1.0 / np.sqrt(_D_MODEL)
    gi = lax.broadcasted_iota(jnp.int32, (_N_GROUPS, _LH), 0)

    def _one_batch(p):
        xb = x_ref[p, :_LH, :]
        tc = tc_ref[p]

        ang = _mm(tc, sel) - ph96
        c96 = jnp.cos(ang)
        te_h = c96[:, :_D_TIME] + c96[:, 2 * _D_TIME:]
        te_q = c96[:, _D_TIME:2 * _D_TIME]

        heads = []
        te_hb, te_qb = _bf(te_h), _bf(te_q)
        for h in range(_N_HEAD):
            cur = None
            for l in range(_N_LAYERS):
                i = h * _N_LAYERS + l
                wq, wk, wv = (wrefs[3 * i][...], wrefs[3 * i + 1][...],
                              wrefs[3 * i + 2][...])
                q = _mm(te_qb, wq[_D_MODEL:])
                if cur is not None:
                    q = q + _mm(_bf(cur), wq[:_D_MODEL])
                k = _mm(xb, wk[:_D_MODEL]) + _mm(te_hb, wk[_D_MODEL:])
                v = _mm(xb, wv[:_D_MODEL]) + _mm(te_hb, wv[_D_MODEL:])
                sc = jnp.where(causal, _mm_t(_bf(q), _bf(k)) * scale, -1e9)
                m = jnp.max(sc, axis=1, keepdims=True)
                pp = jnp.exp(sc - m)
                attn = pp / jnp.sum(pp, axis=1, keepdims=True)
                upd = jnp.tanh(_mm(_bf(attn), _bf(v)))
                cur = upd if cur is None else upd + cur
            heads.append(cur)
        enc = jnp.concatenate(heads, axis=1)

        gid = gid_ref[p][:, :_LH]
        mf = msk_ref[p][:, :_LH]
        oh = jnp.where(gi == gid, mf, 0.0)

        gsum = _mm(oh, enc)
        cnt = jnp.sum(oh, axis=1, keepdims=True)
        inv = 1.0 / jnp.maximum(cnt, 1.0)
        ind = jnp.where(cnt > 0.5, 1.0, 0.0)
        se = se_ref[p]
        re = re_ref[p]
        grep = jnp.concatenate([gsum * inv, ind * se, ind * re], axis=1)

        gp = _mm(grep, gpw_ref[...]) + gpb_ref[...]
        q2 = _mm(gp, mwq_ref[...]) + mbq_ref[...]
        k2 = _mm(gp, mwk_ref[...]) + mbk_ref[...]
        v2 = _mm(gp, mwv_ref[...]) + mbv_ref[...]
        hs = 1.0 / np.sqrt(_MHA_HD)
        outs = []
        for h in range(_GP_DIM // _MHA_HD):
            s = slice(h * _MHA_HD, (h + 1) * _MHA_HD)
            sc2 = _mm_t(q2[:, s], k2[:, s]) * hs
            m2 = jnp.max(sc2, axis=1, keepdims=True)
            p2 = jnp.exp(sc2 - m2)
            a2 = p2 / jnp.sum(p2, axis=1, keepdims=True)
            outs.append(_mm(a2, v2[:, s]))
        att = _mm(jnp.concatenate(outs, axis=1), mwo_ref[...]) + mbo_ref[...]

        def ln(xx, g, bb):
            mu = jnp.mean(xx, axis=1, keepdims=True)
            var = jnp.mean((xx - mu) ** 2, axis=1, keepdims=True)
            return (xx - mu) / jnp.sqrt(var + 1e-5) * g + bb

        gn = ln(gp + att, l1g_ref[...], l1b_ref[...])
        ffn = _mm(jnp.maximum(_mm(gn, fw1_ref[...]) + fb1_ref[...], 0.0),
                  fw2_ref[...]) + fb2_ref[...]
        gout = ln(gn + ffn, l2g_ref[...], l2b_ref[...])

        enhanced = _mm_tl(oh, gout)
        seb = jnp.broadcast_to(se, (_LH, _HIDDEN))
        reb = jnp.broadcast_to(re, (_LH, _HIDDEN))
        merged = jnp.concatenate([enc, seb, reb, enhanced], axis=1)
        return _bf(_mm(_bf(merged), mgw_ref[...]) + mgb_ref[...])

    def _chunk_copy(step, p, j, off, w):
        return pltpu.make_async_copy(
            scr[p * _N_OCHUNK + j],
            out_ref.at[_PAIR * step + p, pl.ds(off, w), :],
            sems.at[p * _N_OCHUNK + j])

    @pl.when(b > 0)
    def _():
        for p in range(_PAIR):
            for j, (off, w) in enumerate(_OCHUNKS):
                _chunk_copy(b - 1, p, j, off, w).wait()

    intw_b = intw_ref[...]
    ib = intb_ref[...]
    for p in range(_PAIR):
        enh_b = _one_batch(p)
        for j, (off, w) in enumerate(_OCHUNKS):
            lg = _mm(enh_b[off:off + w], intw_b) + ib
            scr[p * _N_OCHUNK + j][...] = (
                jnp.maximum(lg, 0.0)
                + jnp.log(1.0 + jnp.exp(-jnp.abs(lg))))
            _chunk_copy(b, p, j, off, w).start()

    @pl.when(b == _NSTEP - 1)
    def _():
        for p in range(_PAIR):
            for j, (off, w) in enumerate(_OCHUNKS):
                _chunk_copy(b, p, j, off, w).wait()


def _full(shape):
    nd = len(shape)
    return pl.BlockSpec(shape, lambda b, _n=nd: (0,) * _n)


def _per_pair(shape):
    nd = len(shape)
    return pl.BlockSpec((_PAIR,) + shape[1:],
                        lambda b, _n=nd: (b,) + (0,) * (_n - 1))


_D_FEAT = 2 * _D_MODEL + 2 * _HIDDEN


def _tc_in_specs():
    specs = [
        _per_pair((_B, _L, _D_MODEL)),
        _per_pair((_B, _LH, 3)),
        _per_pair((_B, 1, _L)),
        _per_pair((_B, 1, _L)),
        _per_pair((_B, 1, _HIDDEN)),
        _per_pair((_B, 1, _HIDDEN)),
    ]
    specs += [_full((_D_MODEL + _D_TIME, _D_MODEL))] * 12
    specs += [
        _full((_D_FEAT, _GP_DIM)), _full((1, _GP_DIM)),
        _full((_GP_DIM, _GP_DIM)), _full((_GP_DIM, _GP_DIM)),
        _full((_GP_DIM, _GP_DIM)), _full((_GP_DIM, _GP_DIM)),
        _full((1, _GP_DIM)), _full((1, _GP_DIM)),
        _full((1, _GP_DIM)), _full((1, _GP_DIM)),
        _full((_GP_DIM, _GP_DIM)), _full((1, _GP_DIM)),
        _full((_GP_DIM, _GP_DIM)), _full((1, _GP_DIM)),
        _full((1, _GP_DIM)), _full((1, _GP_DIM)),
        _full((1, _GP_DIM)), _full((1, _GP_DIM)),
        _full((_D_FEAT + _GP_DIM, _D_FEAT)), _full((1, _D_FEAT)),
        _full((_D_FEAT, _N_ENTITY)), _full((1, _N_ENTITY)),
    ]
    return specs


def _tc_call(*args):
    return pl.pallas_call(
        _tc_body,
        grid=(_NSTEP,),
        in_specs=_tc_in_specs(),
        out_specs=pl.BlockSpec(memory_space=pltpu.MemorySpace.HBM),
        out_shape=jax.ShapeDtypeStruct((_B, _LH, _N_ENTITY), jnp.float32),
        scratch_shapes=(
            [pltpu.VMEM((w, _N_ENTITY), jnp.float32)
             for _ in range(_PAIR) for _, w in _OCHUNKS]
            + [pltpu.SemaphoreType.DMA((_PAIR * _N_OCHUNK,))]),
    )(*args)


def kernel(subs, marks, objs, times, dt, mask, group_map, params):
    subs = subs.astype(jnp.int32)
    marks = marks.astype(jnp.int32)
    objs = objs.astype(jnp.int32)
    group_map = group_map.astype(jnp.int32)

    g_ids, x_rows, s_emb, r_emb = _sc_gather(
        subs.reshape(-1), marks.reshape(-1), objs.reshape(-1),
        group_map, params['event_emb'], params['sub_emb'], params['rel_emb'],
        subs[:, 0], marks[:, 0])

    tcols = jnp.stack([times[:, :-1], times[:, 1:], dt[:, :-1]], axis=-1)
    p = params
    wqkv = []
    for h in range(_N_HEAD):
        for l in range(_N_LAYERS):
            wqkv += [_bf(p[f'Wq_{h}_{l}']), _bf(p[f'Wk_{h}_{l}']),
                     _bf(p[f'Wv_{h}_{l}'])]
    args = (
        _bf(x_rows).reshape(_B, _L, _D_MODEL),
        tcols,
        g_ids.reshape(_B, 1, _L),
        mask.astype(jnp.float32).reshape(_B, 1, _L),
        s_emb.reshape(_B, 1, _HIDDEN),
        r_emb.reshape(_B, 1, _HIDDEN),
        *wqkv,
        p['gp_W'], p['gp_b'].reshape(1, _GP_DIM),
        p['mha_Wq'], p['mha_Wk'], p['mha_Wv'], p['mha_Wo'],
        p['mha_bq'].reshape(1, _GP_DIM), p['mha_bk'].reshape(1, _GP_DIM),
        p['mha_bv'].reshape(1, _GP_DIM), p['mha_bo'].reshape(1, _GP_DIM),
        p['ffn_W1'], p['ffn_b1'].reshape(1, _GP_DIM),
        p['ffn_W2'], p['ffn_b2'].reshape(1, _GP_DIM),
        p['ln1_g'].reshape(1, _GP_DIM), p['ln1_b'].reshape(1, _GP_DIM),
        p['ln2_g'].reshape(1, _GP_DIM), p['ln2_b'].reshape(1, _GP_DIM),
        _bf(p['mg_W']), p['mg_b'].reshape(1, -1),
        _bf(p['int_W']), p['int_b'].reshape(1, -1),
    )
    return _tc_call(*args)


# speedup vs baseline: 1.0575x; 1.0575x over previous
import functools

import numpy as np
import jax
import jax.numpy as jnp
from jax import lax
from jax.experimental import pallas as pl
from jax.experimental.pallas import tpu as pltpu
from jax.experimental.pallas import tpu_sc as plsc

_B, _L = 8, 512
_LH = _L - 1
_N_ENTITY, _N_REL, _N_GROUPS = 2000, 50, 100
_HIDDEN = 128
_D_MODEL, _D_TIME = 128, 32
_N_HEAD, _N_LAYERS = 2, 2
_GP_DIM = 64
_MHA_HD = 32
_NTOK = _B * _L

_NW = 32
_CHUNK = _NTOK // _NW

_OCHUNKS = ((0, 128), (128, 128), (256, 128), (384, 127))
_N_OCHUNK = len(_OCHUNKS)


def _sc_gather_body(subs_hbm, marks_hbm, objs_hbm, gmap_hbm, evemb_hbm,
                    subemb_hbm, relemb_hbm, subs0_hbm, marks0_hbm,
                    gid_out, x_out, semb_out, remb_out,
                    ia_v, ib_v, ic_v, rows_v, idx8_v, rows8_v, sem, sem2):
    wid = lax.axis_index("s") * 2 + lax.axis_index("c")
    base = wid * _CHUNK
    sl = pl.ds(base, _CHUNK)

    pltpu.sync_copy(subs_hbm.at[sl], ia_v)
    pltpu.sync_copy(marks_hbm.at[sl], ib_v)
    pltpu.sync_copy(objs_hbm.at[sl], ic_v)
    for i in range(_CHUNK // 16):
        v = pl.ds(i * 16, 16)
        ib_v[v] = ia_v[v] * _N_REL + ib_v[v]
    cg = pltpu.async_copy(gmap_hbm.at[ib_v], ia_v, sem)
    cr = pltpu.async_copy(evemb_hbm.at[ic_v], rows_v, sem2)
    cg.wait()
    pltpu.sync_copy(ia_v, gid_out.at[sl])
    cr.wait()
    pltpu.sync_copy(rows_v, x_out.at[sl])

    @pl.when(wid == 0)
    def _():
        pltpu.sync_copy(subs0_hbm, idx8_v)
        pltpu.async_copy(subemb_hbm.at[idx8_v], rows8_v, sem).wait()
        pltpu.sync_copy(rows8_v, semb_out)

    @pl.when(wid == 1)
    def _():
        pltpu.sync_copy(marks0_hbm, idx8_v)
        pltpu.async_copy(relemb_hbm.at[idx8_v], rows8_v, sem).wait()
        pltpu.sync_copy(rows8_v, remb_out)


_sc_gather_cache = []


def _sc_gather(*args):
    if not _sc_gather_cache:
        _sc_gather_cache.append(_make_sc_gather())
    return _sc_gather_cache[0](*args)


def _make_sc_gather():
    return functools.partial(
        pl.kernel,
        out_type=(
        jax.ShapeDtypeStruct((_NTOK,), jnp.int32),
        jax.ShapeDtypeStruct((_NTOK, _D_MODEL), jnp.float32),
            jax.ShapeDtypeStruct((_B, _HIDDEN), jnp.float32),
            jax.ShapeDtypeStruct((_B, _HIDDEN), jnp.float32),
        ),
        mesh=plsc.VectorSubcoreMesh(core_axis_name="c", subcore_axis_name="s"),
        scratch_types=(
            pltpu.VMEM((_CHUNK,), jnp.int32),
            pltpu.VMEM((_CHUNK,), jnp.int32),
            pltpu.VMEM((_CHUNK,), jnp.int32),
            pltpu.VMEM((_CHUNK, _D_MODEL), jnp.float32),
            pltpu.VMEM((_B,), jnp.int32),
            pltpu.VMEM((_B, _HIDDEN), jnp.float32),
            pltpu.SemaphoreType.DMA,
            pltpu.SemaphoreType.DMA,
        ),
    )(_sc_gather_body)


def _mm(a, b):
    return lax.dot_general(a, b, (((1,), (0,)), ((), ())),
                           preferred_element_type=jnp.float32)


def _mm_t(a, b):
    return lax.dot_general(a, b, (((1,), (1,)), ((), ())),
                           preferred_element_type=jnp.float32)


def _mm_tl(a, b):
    return lax.dot_general(a, b, (((0,), (0,)), ((), ())),
                           preferred_element_type=jnp.float32)


def _bf(a):
    return a.astype(jnp.bfloat16)


def _mmb(a, b):
    return _mm(_bf(a), _bf(b))


def _mmb_t(a, b):
    return _mm_t(_bf(a), _bf(b))


_PAIR = 1
_NSTEP = _B // _PAIR


def _tc_body(*refs):
    (x_ref, tc_ref, gid_ref, msk_ref, se_ref, re_ref) = refs[:6]
    wrefs = refs[6:18]
    (gpw_ref, gpb_ref,
     mwq_ref, mwk_ref, mwv_ref, mwo_ref,
     mbq_ref, mbk_ref, mbv_ref, mbo_ref,
     fw1_ref, fb1_ref, fw2_ref, fb2_ref,
     l1g_ref, l1b_ref, l2g_ref, l2b_ref,
     mgw_ref, mgb_ref, intw_ref, intb_ref) = refs[18:40]
    out_ref = refs[40]
    scr = refs[41:41 + _PAIR * _N_OCHUNK]
    sems = refs[41 + _PAIR * _N_OCHUNK]
    b = pl.program_id(0)

    half = _D_TIME // 2
    ci3 = lax.broadcasted_iota(jnp.int32, (3, 3 * _D_TIME), 1)
    ri3 = lax.broadcasted_iota(jnp.int32, (3, 3 * _D_TIME), 0)
    kk = ci3 & (_D_TIME - 1)
    k16 = jnp.where(kk < half, kk, kk - half).astype(jnp.float32)
    dvv = jnp.exp(-k16 * (np.log(10000.0) / (half - 1)))
    sel = jnp.where(lax.shift_right_logical(ci3, 5) == ri3, dvv, 0.0)
    ph96 = jnp.where(kk < half, np.float32(np.pi / 2), 0.0)[0:1, :]

    ri = lax.broadcasted_iota(jnp.int32, (_LH, _LH), 0)
    ci = lax.broadcasted_iota(jnp.int32, (_LH, _LH), 1)
    causal = ci <= ri
    scale = 1.0 / np.sqrt(_D_MODEL)
    gi = lax.broadcasted_iota(jnp.int32, (_N_GROUPS, _LH), 0)

    def _one_batch(p):
        xb = _bf(x_ref[p, :_LH, :])
        tc = tc_ref[p]

        ang = _mm(tc, sel) - ph96
        c96 = jnp.cos(ang)
        te_h = c96[:, :_D_TIME] + c96[:, 2 * _D_TIME:]
        te_q = c96[:, _D_TIME:2 * _D_TIME]

        heads = []
        te_hb, te_qb = _bf(te_h), _bf(te_q)
        for h in range(_N_HEAD):
            cur = None
            for l in range(_N_LAYERS):
                i = h * _N_LAYERS + l
                wq, wk, wv = (_bf(wrefs[3 * i][...]),
                              _bf(wrefs[3 * i + 1][...]),
                              _bf(wrefs[3 * i + 2][...]))
                q = _mm(te_qb, wq[_D_MODEL:])
                if cur is not None:
                    q = q + _mm(_bf(cur), wq[:_D_MODEL])
                k = _mm(xb, wk[:_D_MODEL]) + _mm(te_hb, wk[_D_MODEL:])
                v = _mm(xb, wv[:_D_MODEL]) + _mm(te_hb, wv[_D_MODEL:])
                sc = jnp.where(causal, _mm_t(_bf(q), _bf(k)) * scale, -1e9)
                m = jnp.max(sc, axis=1, keepdims=True)
                pp = jnp.exp(sc - m)
                attn = pp / jnp.sum(pp, axis=1, keepdims=True)
                upd = jnp.tanh(_mm(_bf(attn), _bf(v)))
                cur = upd if cur is None else upd + cur
            heads.append(cur)
        enc = jnp.concatenate(heads, axis=1)

        gid = gid_ref[p][:, :_LH]
        mf = msk_ref[p][:, :_LH]
        oh = jnp.where(gi == gid, mf, 0.0)

        gsum = _mm(oh, enc)
        cnt = jnp.sum(oh, axis=1, keepdims=True)
        inv = 1.0 / jnp.maximum(cnt, 1.0)
        ind = jnp.where(cnt > 0.5, 1.0, 0.0)
        se = se_ref[p]
        re = re_ref[p]
        grep = jnp.concatenate([gsum * inv, ind * se, ind * re], axis=1)

        gp = _mm(grep, gpw_ref[...]) + gpb_ref[...]
        q2 = _mm(gp, mwq_ref[...]) + mbq_ref[...]
        k2 = _mm(gp, mwk_ref[...]) + mbk_ref[...]
        v2 = _mm(gp, mwv_ref[...]) + mbv_ref[...]
        hs = 1.0 / np.sqrt(_MHA_HD)
        outs = []
        for h in range(_GP_DIM // _MHA_HD):
            s = slice(h * _MHA_HD, (h + 1) * _MHA_HD)
            sc2 = _mm_t(q2[:, s], k2[:, s]) * hs
            m2 = jnp.max(sc2, axis=1, keepdims=True)
            p2 = jnp.exp(sc2 - m2)
            a2 = p2 / jnp.sum(p2, axis=1, keepdims=True)
            outs.append(_mm(a2, v2[:, s]))
        att = _mm(jnp.concatenate(outs, axis=1), mwo_ref[...]) + mbo_ref[...]

        def ln(xx, g, bb):
            mu = jnp.mean(xx, axis=1, keepdims=True)
            var = jnp.mean((xx - mu) ** 2, axis=1, keepdims=True)
            return (xx - mu) / jnp.sqrt(var + 1e-5) * g + bb

        gn = ln(gp + att, l1g_ref[...], l1b_ref[...])
        ffn = _mm(jnp.maximum(_mm(gn, fw1_ref[...]) + fb1_ref[...], 0.0),
                  fw2_ref[...]) + fb2_ref[...]
        gout = ln(gn + ffn, l2g_ref[...], l2b_ref[...])

        enhanced = _mm_tl(oh, gout)
        seb = jnp.broadcast_to(se, (_LH, _HIDDEN))
        reb = jnp.broadcast_to(re, (_LH, _HIDDEN))
        merged = jnp.concatenate([enc, seb, reb, enhanced], axis=1)
        return _bf(_mmb(merged, mgw_ref[...]) + mgb_ref[...])

    def _chunk_copy(step, p, j, off, w):
        return pltpu.make_async_copy(
            scr[p * _N_OCHUNK + j],
            out_ref.at[_PAIR * step + p, pl.ds(off, w), :],
            sems.at[p * _N_OCHUNK + j])

    @pl.when(b > 0)
    def _():
        for p in range(_PAIR):
            for j, (off, w) in enumerate(_OCHUNKS):
                _chunk_copy(b - 1, p, j, off, w).wait()

    intw_b = _bf(intw_ref[...])
    ib = intb_ref[...]
    for p in range(_PAIR):
        enh_b = _one_batch(p)
        for j, (off, w) in enumerate(_OCHUNKS):
            lg = _mm(enh_b[off:off + w], intw_b) + ib
            scr[p * _N_OCHUNK + j][...] = (
                jnp.maximum(lg, 0.0)
                + jnp.log(1.0 + jnp.exp(-jnp.abs(lg))))
            _chunk_copy(b, p, j, off, w).start()

    @pl.when(b == _NSTEP - 1)
    def _():
        for p in range(_PAIR):
            for j, (off, w) in enumerate(_OCHUNKS):
                _chunk_copy(b, p, j, off, w).wait()


def _full(shape):
    nd = len(shape)
    return pl.BlockSpec(shape, lambda b, _n=nd: (0,) * _n)


def _per_pair(shape):
    nd = len(shape)
    return pl.BlockSpec((_PAIR,) + shape[1:],
                        lambda b, _n=nd: (b,) + (0,) * (_n - 1))


_D_FEAT = 2 * _D_MODEL + 2 * _HIDDEN


def _tc_in_specs():
    specs = [
        _per_pair((_B, _L, _D_MODEL)),
        _per_pair((_B, _LH, 3)),
        _per_pair((_B, 1, _L)),
        _per_pair((_B, 1, _L)),
        _per_pair((_B, 1, _HIDDEN)),
        _per_pair((_B, 1, _HIDDEN)),
    ]
    specs += [_full((_D_MODEL + _D_TIME, _D_MODEL))] * 12
    specs += [
        _full((_D_FEAT, _GP_DIM)), _full((1, _GP_DIM)),
        _full((_GP_DIM, _GP_DIM)), _full((_GP_DIM, _GP_DIM)),
        _full((_GP_DIM, _GP_DIM)), _full((_GP_DIM, _GP_DIM)),
        _full((1, _GP_DIM)), _full((1, _GP_DIM)),
        _full((1, _GP_DIM)), _full((1, _GP_DIM)),
        _full((_GP_DIM, _GP_DIM)), _full((1, _GP_DIM)),
        _full((_GP_DIM, _GP_DIM)), _full((1, _GP_DIM)),
        _full((1, _GP_DIM)), _full((1, _GP_DIM)),
        _full((1, _GP_DIM)), _full((1, _GP_DIM)),
        _full((_D_FEAT + _GP_DIM, _D_FEAT)), _full((1, _D_FEAT)),
        _full((_D_FEAT, _N_ENTITY)), _full((1, _N_ENTITY)),
    ]
    return specs


def _tc_call(*args):
    return pl.pallas_call(
        _tc_body,
        grid=(_NSTEP,),
        in_specs=_tc_in_specs(),
        out_specs=pl.BlockSpec(memory_space=pltpu.MemorySpace.HBM),
        out_shape=jax.ShapeDtypeStruct((_B, _LH, _N_ENTITY), jnp.float32),
        scratch_shapes=(
            [pltpu.VMEM((w, _N_ENTITY), jnp.float32)
             for _ in range(_PAIR) for _, w in _OCHUNKS]
            + [pltpu.SemaphoreType.DMA((_PAIR * _N_OCHUNK,))]),
    )(*args)


def kernel(subs, marks, objs, times, dt, mask, group_map, params):
    subs = subs.astype(jnp.int32)
    marks = marks.astype(jnp.int32)
    objs = objs.astype(jnp.int32)
    group_map = group_map.astype(jnp.int32)

    g_ids, x_rows, s_emb, r_emb = _sc_gather(
        subs.reshape(-1), marks.reshape(-1), objs.reshape(-1),
        group_map, params['event_emb'], params['sub_emb'], params['rel_emb'],
        subs[:, 0], marks[:, 0])

    tcols = jnp.stack([times[:, :-1], times[:, 1:], dt[:, :-1]], axis=-1)
    p = params
    wqkv = []
    for h in range(_N_HEAD):
        for l in range(_N_LAYERS):
            wqkv += [p[f'Wq_{h}_{l}'], p[f'Wk_{h}_{l}'], p[f'Wv_{h}_{l}']]
    args = (
        x_rows.reshape(_B, _L, _D_MODEL),
        tcols,
        g_ids.reshape(_B, 1, _L),
        mask.astype(jnp.float32).reshape(_B, 1, _L),
        s_emb.reshape(_B, 1, _HIDDEN),
        r_emb.reshape(_B, 1, _HIDDEN),
        *wqkv,
        p['gp_W'], p['gp_b'].reshape(1, _GP_DIM),
        p['mha_Wq'], p['mha_Wk'], p['mha_Wv'], p['mha_Wo'],
        p['mha_bq'].reshape(1, _GP_DIM), p['mha_bk'].reshape(1, _GP_DIM),
        p['mha_bv'].reshape(1, _GP_DIM), p['mha_bo'].reshape(1, _GP_DIM),
        p['ffn_W1'], p['ffn_b1'].reshape(1, _GP_DIM),
        p['ffn_W2'], p['ffn_b2'].reshape(1, _GP_DIM),
        p['ln1_g'].reshape(1, _GP_DIM), p['ln1_b'].reshape(1, _GP_DIM),
        p['ln2_g'].reshape(1, _GP_DIM), p['ln2_b'].reshape(1, _GP_DIM),
        p['mg_W'], p['mg_b'].reshape(1, -1),
        p['int_W'], p['int_b'].reshape(1, -1),
    )
    return _tc_call(*args)


# speedup vs baseline: 1.0907x; 1.0313x over previous
import functools

import numpy as np
import jax
import jax.numpy as jnp
from jax import lax
from jax.experimental import pallas as pl
from jax.experimental.pallas import tpu as pltpu
from jax.experimental.pallas import tpu_sc as plsc

_B, _L = 8, 512
_LH = _L - 1
_N_ENTITY, _N_REL, _N_GROUPS = 2000, 50, 100
_HIDDEN = 128
_D_MODEL, _D_TIME = 128, 32
_N_HEAD, _N_LAYERS = 2, 2
_GP_DIM = 64
_MHA_HD = 32
_NTOK = _B * _L

_NW = 32
_CHUNK = _NTOK // _NW

_OCHUNKS = ((0, 128), (128, 128), (256, 128), (384, 127))
_N_OCHUNK = len(_OCHUNKS)


def _sc_gather_body(subs_hbm, marks_hbm, objs_hbm, gmap_hbm, evemb_hbm,
                    subemb_hbm, relemb_hbm, subs0_hbm, marks0_hbm,
                    gid_out, x_out, semb_out, remb_out,
                    ia_v, ib_v, ic_v, rows_v, idx8_v, rows8_v, sem, sem2):
    wid = lax.axis_index("s") * 2 + lax.axis_index("c")
    base = wid * _CHUNK
    sl = pl.ds(base, _CHUNK)

    pltpu.sync_copy(subs_hbm.at[sl], ia_v)
    pltpu.sync_copy(marks_hbm.at[sl], ib_v)
    pltpu.sync_copy(objs_hbm.at[sl], ic_v)
    for i in range(_CHUNK // 16):
        v = pl.ds(i * 16, 16)
        ib_v[v] = ia_v[v] * _N_REL + ib_v[v]
    cg = pltpu.async_copy(gmap_hbm.at[ib_v], ia_v, sem)
    cr = pltpu.async_copy(evemb_hbm.at[ic_v], rows_v, sem2)
    cg.wait()
    pltpu.sync_copy(ia_v, gid_out.at[sl])
    cr.wait()
    pltpu.sync_copy(rows_v, x_out.at[sl])

    @pl.when(wid == 0)
    def _():
        pltpu.sync_copy(subs0_hbm, idx8_v)
        pltpu.async_copy(subemb_hbm.at[idx8_v], rows8_v, sem).wait()
        pltpu.sync_copy(rows8_v, semb_out)

    @pl.when(wid == 1)
    def _():
        pltpu.sync_copy(marks0_hbm, idx8_v)
        pltpu.async_copy(relemb_hbm.at[idx8_v], rows8_v, sem).wait()
        pltpu.sync_copy(rows8_v, remb_out)


_sc_gather_cache = []


def _sc_gather(*args):
    if not _sc_gather_cache:
        _sc_gather_cache.append(_make_sc_gather())
    return _sc_gather_cache[0](*args)


def _make_sc_gather():
    return functools.partial(
        pl.kernel,
        out_type=(
        jax.ShapeDtypeStruct((_NTOK,), jnp.int32),
        jax.ShapeDtypeStruct((_NTOK, _D_MODEL), jnp.float32),
            jax.ShapeDtypeStruct((_B, _HIDDEN), jnp.float32),
            jax.ShapeDtypeStruct((_B, _HIDDEN), jnp.float32),
        ),
        mesh=plsc.VectorSubcoreMesh(core_axis_name="c", subcore_axis_name="s"),
        scratch_types=(
            pltpu.VMEM((_CHUNK,), jnp.int32),
            pltpu.VMEM((_CHUNK,), jnp.int32),
            pltpu.VMEM((_CHUNK,), jnp.int32),
            pltpu.VMEM((_CHUNK, _D_MODEL), jnp.float32),
            pltpu.VMEM((_B,), jnp.int32),
            pltpu.VMEM((_B, _HIDDEN), jnp.float32),
            pltpu.SemaphoreType.DMA,
            pltpu.SemaphoreType.DMA,
        ),
    )(_sc_gather_body)


def _mm(a, b):
    return lax.dot_general(a, b, (((1,), (0,)), ((), ())),
                           preferred_element_type=jnp.float32)


def _mm_t(a, b):
    return lax.dot_general(a, b, (((1,), (1,)), ((), ())),
                           preferred_element_type=jnp.float32)


def _mm_tl(a, b):
    return lax.dot_general(a, b, (((0,), (0,)), ((), ())),
                           preferred_element_type=jnp.float32)


def _bf(a):
    return a.astype(jnp.bfloat16)


def _mmb(a, b):
    return _mm(_bf(a), _bf(b))


def _mmb_t(a, b):
    return _mm_t(_bf(a), _bf(b))


_PAIR = 2
_NSTEP = _B // _PAIR


def _tc_body(*refs):
    (x_ref, tc_ref, gid_ref, msk_ref, se_ref, re_ref) = refs[:6]
    wrefs = refs[6:18]
    (gpw_ref, gpb_ref,
     mwq_ref, mwk_ref, mwv_ref, mwo_ref,
     mbq_ref, mbk_ref, mbv_ref, mbo_ref,
     fw1_ref, fb1_ref, fw2_ref, fb2_ref,
     l1g_ref, l1b_ref, l2g_ref, l2b_ref,
     mgw_ref, mgb_ref, intw_ref, intb_ref) = refs[18:40]
    out_ref = refs[40]
    nscr = 41 + _PAIR * _N_OCHUNK
    scr = refs[41:nscr]
    sems = refs[nscr]
    b = pl.program_id(0)

    half = _D_TIME // 2
    ci3 = lax.broadcasted_iota(jnp.int32, (3, 3 * _D_TIME), 1)
    ri3 = lax.broadcasted_iota(jnp.int32, (3, 3 * _D_TIME), 0)
    kk = ci3 & (_D_TIME - 1)
    k16 = jnp.where(kk < half, kk, kk - half).astype(jnp.float32)
    dvv = jnp.exp(-k16 * (np.log(10000.0) / (half - 1)))
    sel = jnp.where(lax.shift_right_logical(ci3, 5) == ri3, dvv, 0.0)
    ph96 = jnp.where(kk < half, np.float32(np.pi / 2), 0.0)[0:1, :]

    ri = lax.broadcasted_iota(jnp.int32, (_LH, _LH), 0)
    ci = lax.broadcasted_iota(jnp.int32, (_LH, _LH), 1)
    causal = ci <= ri
    scale = 1.0 / np.sqrt(_D_MODEL)
    gi = lax.broadcasted_iota(jnp.int32, (_N_GROUPS, _LH), 0)

    def _one_batch(p):
        xb = _bf(x_ref[p, :_LH, :])
        tc = tc_ref[p]

        ang = _mm(tc, sel) - ph96
        c96 = jnp.cos(ang)
        te_h = c96[:, :_D_TIME] + c96[:, 2 * _D_TIME:]
        te_q = c96[:, _D_TIME:2 * _D_TIME]

        heads = []
        te_hb, te_qb = _bf(te_h), _bf(te_q)
        for h in range(_N_HEAD):
            cur = None
            for l in range(_N_LAYERS):
                i = h * _N_LAYERS + l
                wq, wk, wv = (_bf(wrefs[3 * i][...]),
                              _bf(wrefs[3 * i + 1][...]),
                              _bf(wrefs[3 * i + 2][...]))
                q = _mm(te_qb, wq[_D_MODEL:])
                if cur is not None:
                    q = q + _mm(_bf(cur), wq[:_D_MODEL])
                k = _mm(xb, wk[:_D_MODEL]) + _mm(te_hb, wk[_D_MODEL:])
                v = _mm(xb, wv[:_D_MODEL]) + _mm(te_hb, wv[_D_MODEL:])
                sc = jnp.where(causal, _mm_t(_bf(q), _bf(k)) * scale, -1e9)
                m = jnp.max(sc, axis=1, keepdims=True)
                pp = jnp.exp(sc - m)
                attn = pp / jnp.sum(pp, axis=1, keepdims=True)
                upd = jnp.tanh(_mm(_bf(attn), _bf(v)))
                cur = upd if cur is None else upd + cur
            heads.append(cur)
        enc = jnp.concatenate(heads, axis=1)

        gid = gid_ref[p][:, :_LH]
        mf = msk_ref[p][:, :_LH]
        oh = jnp.where(gi == gid, mf, 0.0)

        gsum = _mm(oh, enc)
        cnt = jnp.sum(oh, axis=1, keepdims=True)
        inv = 1.0 / jnp.maximum(cnt, 1.0)
        ind = jnp.where(cnt > 0.5, 1.0, 0.0)
        se = se_ref[p]
        re = re_ref[p]
        grep = jnp.concatenate([gsum * inv, ind * se, ind * re], axis=1)

        gp = _mm(grep, gpw_ref[...]) + gpb_ref[...]
        q2 = _mm(gp, mwq_ref[...]) + mbq_ref[...]
        k2 = _mm(gp, mwk_ref[...]) + mbk_ref[...]
        v2 = _mm(gp, mwv_ref[...]) + mbv_ref[...]
        hs = 1.0 / np.sqrt(_MHA_HD)
        outs = []
        for h in range(_GP_DIM // _MHA_HD):
            s = slice(h * _MHA_HD, (h + 1) * _MHA_HD)
            sc2 = _mm_t(q2[:, s], k2[:, s]) * hs
            m2 = jnp.max(sc2, axis=1, keepdims=True)
            p2 = jnp.exp(sc2 - m2)
            a2 = p2 / jnp.sum(p2, axis=1, keepdims=True)
            outs.append(_mm(a2, v2[:, s]))
        att = _mm(jnp.concatenate(outs, axis=1), mwo_ref[...]) + mbo_ref[...]

        def ln(xx, g, bb):
            mu = jnp.mean(xx, axis=1, keepdims=True)
            var = jnp.mean((xx - mu) ** 2, axis=1, keepdims=True)
            return (xx - mu) / jnp.sqrt(var + 1e-5) * g + bb

        gn = ln(gp + att, l1g_ref[...], l1b_ref[...])
        ffn = _mm(jnp.maximum(_mm(gn, fw1_ref[...]) + fb1_ref[...], 0.0),
                  fw2_ref[...]) + fb2_ref[...]
        gout = ln(gn + ffn, l2g_ref[...], l2b_ref[...])

        enhanced = _mm_tl(oh, gout)
        seb = jnp.broadcast_to(se, (_LH, _HIDDEN))
        reb = jnp.broadcast_to(re, (_LH, _HIDDEN))
        merged = jnp.concatenate([enc, seb, reb, enhanced], axis=1)
        return _bf(_mmb(merged, mgw_ref[...]) + mgb_ref[...])

    def _chunk_copy(step, p, j, off, w):
        return pltpu.make_async_copy(
            scr[p * _N_OCHUNK + j],
            out_ref.at[_PAIR * step + p, pl.ds(off, w), :],
            sems.at[p * _N_OCHUNK + j])

    @pl.when(b > 0)
    def _():
        for p in range(_PAIR):
            for j, (off, w) in enumerate(_OCHUNKS):
                _chunk_copy(b - 1, p, j, off, w).wait()

    intw_b = _bf(intw_ref[...])
    ib = intb_ref[...]
    for p in range(_PAIR):
        enh_b = _one_batch(p)
        for j, (off, w) in enumerate(_OCHUNKS):
            lg = _mm(enh_b[off:off + w], intw_b) + ib
            scr[p * _N_OCHUNK + j][...] = (
                jnp.maximum(lg, 0.0)
                + jnp.log(1.0 + jnp.exp(-jnp.abs(lg))))
            _chunk_copy(b, p, j, off, w).start()

    @pl.when(b == _NSTEP - 1)
    def _():
        for p in range(_PAIR):
            for j, (off, w) in enumerate(_OCHUNKS):
                _chunk_copy(b, p, j, off, w).wait()


def _full(shape):
    nd = len(shape)
    return pl.BlockSpec(shape, lambda b, _n=nd: (0,) * _n)


def _per_pair(shape):
    nd = len(shape)
    return pl.BlockSpec((_PAIR,) + shape[1:],
                        lambda b, _n=nd: (b,) + (0,) * (_n - 1))


_D_FEAT = 2 * _D_MODEL + 2 * _HIDDEN


def _tc_in_specs():
    specs = [
        _per_pair((_B, _L, _D_MODEL)),
        _per_pair((_B, _LH, 3)),
        _per_pair((_B, 1, _L)),
        _per_pair((_B, 1, _L)),
        _per_pair((_B, 1, _HIDDEN)),
        _per_pair((_B, 1, _HIDDEN)),
    ]
    specs += [_full((_D_MODEL + _D_TIME, _D_MODEL))] * 12
    specs += [
        _full((_D_FEAT, _GP_DIM)), _full((1, _GP_DIM)),
        _full((_GP_DIM, _GP_DIM)), _full((_GP_DIM, _GP_DIM)),
        _full((_GP_DIM, _GP_DIM)), _full((_GP_DIM, _GP_DIM)),
        _full((1, _GP_DIM)), _full((1, _GP_DIM)),
        _full((1, _GP_DIM)), _full((1, _GP_DIM)),
        _full((_GP_DIM, _GP_DIM)), _full((1, _GP_DIM)),
        _full((_GP_DIM, _GP_DIM)), _full((1, _GP_DIM)),
        _full((1, _GP_DIM)), _full((1, _GP_DIM)),
        _full((1, _GP_DIM)), _full((1, _GP_DIM)),
        _full((_D_FEAT + _GP_DIM, _D_FEAT)), _full((1, _D_FEAT)),
        _full((_D_FEAT, _N_ENTITY)), _full((1, _N_ENTITY)),
    ]
    return specs


def _tc_call(*args):
    return pl.pallas_call(
        _tc_body,
        grid=(_NSTEP,),
        in_specs=_tc_in_specs(),
        out_specs=pl.BlockSpec(memory_space=pltpu.MemorySpace.HBM),
        out_shape=jax.ShapeDtypeStruct((_B, _LH, _N_ENTITY), jnp.float32),
        scratch_shapes=(
            [pltpu.VMEM((w, _N_ENTITY), jnp.float32)
             for _ in range(_PAIR) for _, w in _OCHUNKS]
            + [pltpu.SemaphoreType.DMA((_PAIR * _N_OCHUNK,))]),
    )(*args)


def kernel(subs, marks, objs, times, dt, mask, group_map, params):
    subs = subs.astype(jnp.int32)
    marks = marks.astype(jnp.int32)
    objs = objs.astype(jnp.int32)
    group_map = group_map.astype(jnp.int32)

    g_ids, x_rows, s_emb, r_emb = _sc_gather(
        subs.reshape(-1), marks.reshape(-1), objs.reshape(-1),
        group_map, params['event_emb'], params['sub_emb'], params['rel_emb'],
        subs[:, 0], marks[:, 0])

    tcols = jnp.stack([times[:, :-1], times[:, 1:], dt[:, :-1]], axis=-1)
    p = params
    wqkv = []
    for h in range(_N_HEAD):
        for l in range(_N_LAYERS):
            wqkv += [p[f'Wq_{h}_{l}'], p[f'Wk_{h}_{l}'], p[f'Wv_{h}_{l}']]
    args = (
        x_rows.reshape(_B, _L, _D_MODEL),
        tcols,
        g_ids.reshape(_B, 1, _L),
        mask.astype(jnp.float32).reshape(_B, 1, _L),
        s_emb.reshape(_B, 1, _HIDDEN),
        r_emb.reshape(_B, 1, _HIDDEN),
        *wqkv,
        p['gp_W'], p['gp_b'].reshape(1, _GP_DIM),
        p['mha_Wq'], p['mha_Wk'], p['mha_Wv'], p['mha_Wo'],
        p['mha_bq'].reshape(1, _GP_DIM), p['mha_bk'].reshape(1, _GP_DIM),
        p['mha_bv'].reshape(1, _GP_DIM), p['mha_bo'].reshape(1, _GP_DIM),
        p['ffn_W1'], p['ffn_b1'].reshape(1, _GP_DIM),
        p['ffn_W2'], p['ffn_b2'].reshape(1, _GP_DIM),
        p['ln1_g'].reshape(1, _GP_DIM), p['ln1_b'].reshape(1, _GP_DIM),
        p['ln2_g'].reshape(1, _GP_DIM), p['ln2_b'].reshape(1, _GP_DIM),
        p['mg_W'], p['mg_b'].reshape(1, -1),
        p['int_W'], p['int_b'].reshape(1, -1),
    )
    return _tc_call(*args)


# speedup vs baseline: 1.0955x; 1.0045x over previous
import functools

import numpy as np
import jax
import jax.numpy as jnp
from jax import lax
from jax.experimental import pallas as pl
from jax.experimental.pallas import tpu as pltpu
from jax.experimental.pallas import tpu_sc as plsc

_B, _L = 8, 512
_LH = _L - 1
_N_ENTITY, _N_REL, _N_GROUPS = 2000, 50, 100
_HIDDEN = 128
_D_MODEL, _D_TIME = 128, 32
_N_HEAD, _N_LAYERS = 2, 2
_GP_DIM = 64
_MHA_HD = 32
_NTOK = _B * _L

_NW = 32
_CHUNK = _NTOK // _NW

_OCHUNKS = ((0, 128), (128, 128), (256, 128), (384, 127))
_N_OCHUNK = len(_OCHUNKS)


def _sc_gather_body(subs_hbm, marks_hbm, objs_hbm, gmap_hbm, evemb_hbm,
                    subemb_hbm, relemb_hbm, subs0_hbm, marks0_hbm,
                    gid_out, x_out, semb_out, remb_out,
                    ia_v, ib_v, ic_v, rows_v, idx8_v, rows8_v, sem, sem2):
    wid = lax.axis_index("s") * 2 + lax.axis_index("c")
    base = wid * _CHUNK
    sl = pl.ds(base, _CHUNK)

    pltpu.sync_copy(subs_hbm.at[sl], ia_v)
    pltpu.sync_copy(marks_hbm.at[sl], ib_v)
    pltpu.sync_copy(objs_hbm.at[sl], ic_v)
    for i in range(_CHUNK // 16):
        v = pl.ds(i * 16, 16)
        ib_v[v] = ia_v[v] * _N_REL + ib_v[v]
    cg = pltpu.async_copy(gmap_hbm.at[ib_v], ia_v, sem)
    cr = pltpu.async_copy(evemb_hbm.at[ic_v], rows_v, sem2)
    cg.wait()
    pltpu.sync_copy(ia_v, gid_out.at[sl])
    cr.wait()
    pltpu.sync_copy(rows_v, x_out.at[sl])

    @pl.when(wid == 0)
    def _():
        pltpu.sync_copy(subs0_hbm, idx8_v)
        pltpu.async_copy(subemb_hbm.at[idx8_v], rows8_v, sem).wait()
        pltpu.sync_copy(rows8_v, semb_out)

    @pl.when(wid == 1)
    def _():
        pltpu.sync_copy(marks0_hbm, idx8_v)
        pltpu.async_copy(relemb_hbm.at[idx8_v], rows8_v, sem).wait()
        pltpu.sync_copy(rows8_v, remb_out)


_sc_gather_cache = []


def _sc_gather(*args):
    if not _sc_gather_cache:
        _sc_gather_cache.append(_make_sc_gather())
    return _sc_gather_cache[0](*args)


def _make_sc_gather():
    return functools.partial(
        pl.kernel,
        out_type=(
        jax.ShapeDtypeStruct((_NTOK,), jnp.int32),
        jax.ShapeDtypeStruct((_NTOK, _D_MODEL), jnp.float32),
            jax.ShapeDtypeStruct((_B, _HIDDEN), jnp.float32),
            jax.ShapeDtypeStruct((_B, _HIDDEN), jnp.float32),
        ),
        mesh=plsc.VectorSubcoreMesh(core_axis_name="c", subcore_axis_name="s"),
        scratch_types=(
            pltpu.VMEM((_CHUNK,), jnp.int32),
            pltpu.VMEM((_CHUNK,), jnp.int32),
            pltpu.VMEM((_CHUNK,), jnp.int32),
            pltpu.VMEM((_CHUNK, _D_MODEL), jnp.float32),
            pltpu.VMEM((_B,), jnp.int32),
            pltpu.VMEM((_B, _HIDDEN), jnp.float32),
            pltpu.SemaphoreType.DMA,
            pltpu.SemaphoreType.DMA,
        ),
    )(_sc_gather_body)


def _mm(a, b):
    return lax.dot_general(a, b, (((1,), (0,)), ((), ())),
                           preferred_element_type=jnp.float32)


def _mm_t(a, b):
    return lax.dot_general(a, b, (((1,), (1,)), ((), ())),
                           preferred_element_type=jnp.float32)


def _mm_tl(a, b):
    return lax.dot_general(a, b, (((0,), (0,)), ((), ())),
                           preferred_element_type=jnp.float32)


def _bf(a):
    return a.astype(jnp.bfloat16)


def _mmb(a, b):
    return _mm(_bf(a), _bf(b))


def _mmb_t(a, b):
    return _mm_t(_bf(a), _bf(b))


_PAIR = 2
_NSTEP = _B // _PAIR


def _tc_body(*refs):
    (x_ref, tm_ref, dt_ref, gid_ref, msk_ref, se_ref, re_ref) = refs[:7]
    wrefs = refs[7:19]
    (gpw_ref, gpb_ref,
     mwq_ref, mwk_ref, mwv_ref, mwo_ref,
     mbq_ref, mbk_ref, mbv_ref, mbo_ref,
     fw1_ref, fb1_ref, fw2_ref, fb2_ref,
     l1g_ref, l1b_ref, l2g_ref, l2b_ref,
     mgw_ref, mgb_ref, intw_ref, intb_ref) = refs[19:41]
    out_ref = refs[41]
    nscr = 42 + _PAIR * _N_OCHUNK
    scr = refs[42:nscr]
    sems = refs[nscr]
    b = pl.program_id(0)

    half = _D_TIME // 2
    ci3 = lax.broadcasted_iota(jnp.int32, (3, 3 * _D_TIME), 1)
    ri3 = lax.broadcasted_iota(jnp.int32, (3, 3 * _D_TIME), 0)
    kk = ci3 & (_D_TIME - 1)
    k16 = jnp.where(kk < half, kk, kk - half).astype(jnp.float32)
    dvv = jnp.exp(-k16 * (np.log(10000.0) / (half - 1)))
    sel = jnp.where(lax.shift_right_logical(ci3, 5) == ri3, dvv, 0.0)
    ph96 = jnp.where(kk < half, np.float32(np.pi / 2), 0.0)[0:1, :]

    ri = lax.broadcasted_iota(jnp.int32, (_LH, _LH), 0)
    ci = lax.broadcasted_iota(jnp.int32, (_LH, _LH), 1)
    causal = ci <= ri
    scale = 1.0 / np.sqrt(_D_MODEL)
    gi = lax.broadcasted_iota(jnp.int32, (_N_GROUPS, _LH), 0)

    def _one_batch(p):
        xb = _bf(x_ref[p, :_LH, :])
        tc3 = jnp.concatenate(
            [tm_ref[p, :, :_LH], tm_ref[p, :, 1:_L], dt_ref[p, :, :_LH]],
            axis=0)

        ang = _mm_tl(tc3, sel) - ph96
        c96 = jnp.cos(ang)
        te_h = c96[:, :_D_TIME] + c96[:, 2 * _D_TIME:]
        te_q = c96[:, _D_TIME:2 * _D_TIME]

        heads = []
        te_hb, te_qb = _bf(te_h), _bf(te_q)
        for h in range(_N_HEAD):
            cur = None
            for l in range(_N_LAYERS):
                i = h * _N_LAYERS + l
                wq, wk, wv = (_bf(wrefs[3 * i][...]),
                              _bf(wrefs[3 * i + 1][...]),
                              _bf(wrefs[3 * i + 2][...]))
                q = _mm(te_qb, wq[_D_MODEL:])
                if cur is not None:
                    q = q + _mm(_bf(cur), wq[:_D_MODEL])
                k = _mm(xb, wk[:_D_MODEL]) + _mm(te_hb, wk[_D_MODEL:])
                v = _mm(xb, wv[:_D_MODEL]) + _mm(te_hb, wv[_D_MODEL:])
                sc = jnp.where(causal, _mm_t(_bf(q), _bf(k)) * scale, -1e9)
                m = jnp.max(sc, axis=1, keepdims=True)
                pp = jnp.exp(sc - m)
                attn = pp / jnp.sum(pp, axis=1, keepdims=True)
                upd = jnp.tanh(_mm(_bf(attn), _bf(v)))
                cur = upd if cur is None else upd + cur
            heads.append(cur)
        enc = jnp.concatenate(heads, axis=1)

        gid = gid_ref[p][:, :_LH]
        mb = msk_ref[p][:, :_LH]
        oh = jnp.where((gi == gid) & mb, 1.0, 0.0)

        gsum = _mm(oh, enc)
        cnt = jnp.sum(oh, axis=1, keepdims=True)
        inv = 1.0 / jnp.maximum(cnt, 1.0)
        ind = jnp.where(cnt > 0.5, 1.0, 0.0)
        se = se_ref[p]
        re = re_ref[p]
        grep = jnp.concatenate([gsum * inv, ind * se, ind * re], axis=1)

        gp = _mm(grep, gpw_ref[...]) + gpb_ref[...]
        q2 = _mm(gp, mwq_ref[...]) + mbq_ref[...]
        k2 = _mm(gp, mwk_ref[...]) + mbk_ref[...]
        v2 = _mm(gp, mwv_ref[...]) + mbv_ref[...]
        hs = 1.0 / np.sqrt(_MHA_HD)
        outs = []
        for h in range(_GP_DIM // _MHA_HD):
            s = slice(h * _MHA_HD, (h + 1) * _MHA_HD)
            sc2 = _mm_t(q2[:, s], k2[:, s]) * hs
            m2 = jnp.max(sc2, axis=1, keepdims=True)
            p2 = jnp.exp(sc2 - m2)
            a2 = p2 / jnp.sum(p2, axis=1, keepdims=True)
            outs.append(_mm(a2, v2[:, s]))
        att = _mm(jnp.concatenate(outs, axis=1), mwo_ref[...]) + mbo_ref[...]

        def ln(xx, g, bb):
            mu = jnp.mean(xx, axis=1, keepdims=True)
            var = jnp.mean((xx - mu) ** 2, axis=1, keepdims=True)
            return (xx - mu) / jnp.sqrt(var + 1e-5) * g + bb

        gn = ln(gp + att, l1g_ref[...], l1b_ref[...])
        ffn = _mm(jnp.maximum(_mm(gn, fw1_ref[...]) + fb1_ref[...], 0.0),
                  fw2_ref[...]) + fb2_ref[...]
        gout = ln(gn + ffn, l2g_ref[...], l2b_ref[...])

        enhanced = _mm_tl(oh, gout)
        seb = jnp.broadcast_to(se, (_LH, _HIDDEN))
        reb = jnp.broadcast_to(re, (_LH, _HIDDEN))
        merged = jnp.concatenate([enc, seb, reb, enhanced], axis=1)
        return _bf(_mmb(merged, mgw_ref[...]) + mgb_ref[...])

    def _chunk_copy(step, p, j, off, w):
        return pltpu.make_async_copy(
            scr[p * _N_OCHUNK + j],
            out_ref.at[_PAIR * step + p, pl.ds(off, w), :],
            sems.at[p * _N_OCHUNK + j])

    @pl.when(b > 0)
    def _():
        for p in range(_PAIR):
            for j, (off, w) in enumerate(_OCHUNKS):
                _chunk_copy(b - 1, p, j, off, w).wait()

    intw_b = _bf(intw_ref[...])
    ib = intb_ref[...]
    for p in range(_PAIR):
        enh_b = _one_batch(p)
        for j, (off, w) in enumerate(_OCHUNKS):
            lg = _mm(enh_b[off:off + w], intw_b) + ib
            scr[p * _N_OCHUNK + j][...] = (
                jnp.maximum(lg, 0.0)
                + jnp.log(1.0 + jnp.exp(-jnp.abs(lg))))
            _chunk_copy(b, p, j, off, w).start()

    @pl.when(b == _NSTEP - 1)
    def _():
        for p in range(_PAIR):
            for j, (off, w) in enumerate(_OCHUNKS):
                _chunk_copy(b, p, j, off, w).wait()


def _full(shape):
    nd = len(shape)
    return pl.BlockSpec(shape, lambda b, _n=nd: (0,) * _n)


def _per_pair(shape):
    nd = len(shape)
    return pl.BlockSpec((_PAIR,) + shape[1:],
                        lambda b, _n=nd: (b,) + (0,) * (_n - 1))


_D_FEAT = 2 * _D_MODEL + 2 * _HIDDEN


def _tc_in_specs():
    specs = [
        _per_pair((_B, _L, _D_MODEL)),
        _per_pair((_B, 1, _L)),
        _per_pair((_B, 1, _L)),
        _per_pair((_B, 1, _L)),
        _per_pair((_B, 1, _L)),
        _per_pair((_B, 1, _HIDDEN)),
        _per_pair((_B, 1, _HIDDEN)),
    ]
    specs += [_full((_D_MODEL + _D_TIME, _D_MODEL))] * 12
    specs += [
        _full((_D_FEAT, _GP_DIM)), _full((1, _GP_DIM)),
        _full((_GP_DIM, _GP_DIM)), _full((_GP_DIM, _GP_DIM)),
        _full((_GP_DIM, _GP_DIM)), _full((_GP_DIM, _GP_DIM)),
        _full((1, _GP_DIM)), _full((1, _GP_DIM)),
        _full((1, _GP_DIM)), _full((1, _GP_DIM)),
        _full((_GP_DIM, _GP_DIM)), _full((1, _GP_DIM)),
        _full((_GP_DIM, _GP_DIM)), _full((1, _GP_DIM)),
        _full((1, _GP_DIM)), _full((1, _GP_DIM)),
        _full((1, _GP_DIM)), _full((1, _GP_DIM)),
        _full((_D_FEAT + _GP_DIM, _D_FEAT)), _full((1, _D_FEAT)),
        _full((_D_FEAT, _N_ENTITY)), _full((1, _N_ENTITY)),
    ]
    return specs


def _tc_call(*args):
    return pl.pallas_call(
        _tc_body,
        grid=(_NSTEP,),
        in_specs=_tc_in_specs(),
        out_specs=pl.BlockSpec(memory_space=pltpu.MemorySpace.HBM),
        out_shape=jax.ShapeDtypeStruct((_B, _LH, _N_ENTITY), jnp.float32),
        scratch_shapes=(
            [pltpu.VMEM((w, _N_ENTITY), jnp.float32)
             for _ in range(_PAIR) for _, w in _OCHUNKS]
            + [pltpu.SemaphoreType.DMA((_PAIR * _N_OCHUNK,))]),
    )(*args)


def kernel(subs, marks, objs, times, dt, mask, group_map, params):
    subs = subs.astype(jnp.int32)
    marks = marks.astype(jnp.int32)
    objs = objs.astype(jnp.int32)
    group_map = group_map.astype(jnp.int32)

    g_ids, x_rows, s_emb, r_emb = _sc_gather(
        subs.reshape(-1), marks.reshape(-1), objs.reshape(-1),
        group_map, params['event_emb'], params['sub_emb'], params['rel_emb'],
        subs[:, 0], marks[:, 0])

    p = params
    wqkv = []
    for h in range(_N_HEAD):
        for l in range(_N_LAYERS):
            wqkv += [p[f'Wq_{h}_{l}'], p[f'Wk_{h}_{l}'], p[f'Wv_{h}_{l}']]
    args = (
        x_rows.reshape(_B, _L, _D_MODEL),
        times.reshape(_B, 1, _L),
        dt.reshape(_B, 1, _L),
        g_ids.reshape(_B, 1, _L),
        mask.reshape(_B, 1, _L),
        s_emb.reshape(_B, 1, _HIDDEN),
        r_emb.reshape(_B, 1, _HIDDEN),
        *wqkv,
        p['gp_W'], p['gp_b'].reshape(1, _GP_DIM),
        p['mha_Wq'], p['mha_Wk'], p['mha_Wv'], p['mha_Wo'],
        p['mha_bq'].reshape(1, _GP_DIM), p['mha_bk'].reshape(1, _GP_DIM),
        p['mha_bv'].reshape(1, _GP_DIM), p['mha_bo'].reshape(1, _GP_DIM),
        p['ffn_W1'], p['ffn_b1'].reshape(1, _GP_DIM),
        p['ffn_W2'], p['ffn_b2'].reshape(1, _GP_DIM),
        p['ln1_g'].reshape(1, _GP_DIM), p['ln1_b'].reshape(1, _GP_DIM),
        p['ln2_g'].reshape(1, _GP_DIM), p['ln2_b'].reshape(1, _GP_DIM),
        p['mg_W'], p['mg_b'].reshape(1, -1),
        p['int_W'], p['int_b'].reshape(1, -1),
    )
    return _tc_call(*args)
